# baseline (speedup 1.0000x reference)
import functools

import jax
import jax.numpy as jnp
from jax import lax
from jax.experimental import pallas as pl
from jax.experimental.pallas import tpu as pltpu
from jax.experimental.pallas import tpu_sc as plsc

N = 10000
E = 320000
D = 128
H = 16
C = 7

NC = 2
NS = 16
NW = NC * NS
EW = E // NW
CHUNK = 2000
NCHUNK = EW // CHUNK
NBUF = 3
NPAD = 10240
RPT = NPAD // NS

_mesh = plsc.VectorSubcoreMesh(core_axis_name="c", subcore_axis_name="s")
_sc_params = pltpu.CompilerParams(use_tc_tiling_on_sc=False)


def _fill(ref, n, value):
    v = jnp.full((16,), value, ref.dtype)

    def body(i, _):
        ref[pl.ds(i * 16, 16)] = v
        return 0

    lax.fori_loop(0, n // 16, body, 0)


@functools.partial(
    pl.kernel,
    out_type=jax.ShapeDtypeStruct((NC, NPAD, H), jnp.float32),
    mesh=_mesh,
    scratch_types=[
        pltpu.VMEM((EW,), jnp.int32),
        pltpu.VMEM((EW,), jnp.float32),
        pltpu.VMEM((RPT,), jnp.float32),
        pltpu.VMEM((RPT, H), jnp.float32),
        pltpu.VMEM_SHARED((NPAD,), jnp.float32),
        pltpu.SemaphoreType.DMA,
    ],
    compiler_params=_sc_params,
)
def _deg_kernel(dst_hbm, out_hbm, idx_v, ones_v, buf_v, ebuf_v, deg_sh, sem):
    c = lax.axis_index("c")
    s = lax.axis_index("s")
    wid = s * NC + c
    ld = pltpu.async_copy(dst_hbm.at[pl.ds(wid * EW, EW)], idx_v, sem)
    _fill(ones_v, EW, 1.0)
    _fill(buf_v, RPT, 0.0)
    pltpu.sync_copy(buf_v, deg_sh.at[pl.ds(s * RPT, RPT)])
    plsc.subcore_barrier()

    ld.wait()
    pltpu.sync_copy(ones_v, deg_sh.at[idx_v], add=True)

    plsc.subcore_barrier()
    pltpu.sync_copy(deg_sh.at[pl.ds(s * RPT, RPT)], buf_v)

    def expand(g, _):
        vec = buf_v[pl.ds(g * 16, 16)]
        for i in range(16):
            ebuf_v[g * 16 + i, :] = jnp.broadcast_to(vec[i], (16,))
        return 0

    lax.fori_loop(0, RPT // 16, expand, 0)
    pltpu.sync_copy(ebuf_v, out_hbm.at[c, pl.ds(s * RPT, RPT)])


@functools.partial(
    pl.kernel,
    out_type=jax.ShapeDtypeStruct((NC, NPAD, H), jnp.float32),
    mesh=_mesh,
    scratch_types=(
        [pltpu.VMEM((CHUNK,), jnp.int32) for _ in range(NBUF)]
        + [pltpu.VMEM((CHUNK,), jnp.int32) for _ in range(NBUF)]
        + [pltpu.VMEM((CHUNK, H), jnp.float32) for _ in range(NBUF)]
        + [
            pltpu.VMEM((RPT, H), jnp.float32),
            pltpu.VMEM_SHARED((NPAD, H), jnp.float32),
        ]
        + [pltpu.SemaphoreType.DMA for _ in range(3)]
        + [pltpu.SemaphoreType.DMA for _ in range(2)]
        + [pltpu.SemaphoreType.DMA for _ in range(2)]
    ),
    compiler_params=_sc_params,
)
def _agg_kernel(y_hbm, edge_hbm, out_hbm, *refs):
    c = lax.axis_index("c")
    s = lax.axis_index("s")
    wid = s * NC + c
    sidx = refs[0:NBUF]
    didx = refs[NBUF:2 * NBUF]
    rows = refs[2 * NBUF:3 * NBUF]
    buf_v, acc_sh = refs[3 * NBUF], refs[3 * NBUF + 1]
    semI = refs[3 * NBUF + 2:3 * NBUF + 5]
    semG = refs[3 * NBUF + 5:3 * NBUF + 7]
    semS = refs[3 * NBUF + 7:3 * NBUF + 9]

    base = wid * EW

    def load_idx(j):
        sem = semI[j % 3]
        b = j % NBUF
        a1 = pltpu.async_copy(
            edge_hbm.at[0, pl.ds(base + j * CHUNK, CHUNK)], sidx[b], sem)
        a2 = pltpu.async_copy(
            edge_hbm.at[1, pl.ds(base + j * CHUNK, CHUNK)], didx[b], sem)
        return (a1, a2)

    def gather(j):
        return pltpu.async_copy(
            y_hbm.at[sidx[j % NBUF]], rows[j % NBUF], semG[j % 2])

    def scatter(j):
        return pltpu.async_copy(
            rows[j % NBUF], acc_sh.at[didx[j % NBUF]], semS[j % 2], add=True)

    ld = load_idx(0)
    pend_ld = load_idx(1)

    def fill_zero(i, _):
        buf_v[i, :] = jnp.zeros((16,), jnp.float32)
        return 0

    lax.fori_loop(0, RPT, fill_zero, 0)
    pltpu.sync_copy(buf_v, acc_sh.at[pl.ds(s * RPT, RPT)])
    plsc.subcore_barrier()

    ld[0].wait()
    ld[1].wait()
    g = gather(0)
    pend_sc = {}
    for j in range(NCHUNK):
        g.wait()
        pend_sc[j] = scatter(j)
        if j + 2 < NCHUNK:
            if j - 1 >= 0:
                pend_sc.pop(j - 1).wait()
            nxt = load_idx(j + 2)
        if j + 1 < NCHUNK:
            pend_ld[0].wait()
            pend_ld[1].wait()
            g = gather(j + 1)
            pend_ld = nxt
    for j in sorted(pend_sc):
        pend_sc[j].wait()

    plsc.subcore_barrier()
    pltpu.sync_copy(acc_sh.at[pl.ds(s * RPT, RPT)], buf_v)
    pltpu.sync_copy(buf_v, out_hbm.at[c, pl.ds(s * RPT, RPT)])


NP8 = N // 8
NPAD8 = NPAD // 8


def _dinvp(degp_ref):
    deg = degp_ref[0, :NP8, :] + degp_ref[1, :NP8, :] + 1.0
    return lax.rsqrt(deg)


def _mm_body(xf_ref, w1bd_ref, xw_ref):
    xw_ref[...] = jnp.dot(xf_ref[...], w1bd_ref[...],
                          preferred_element_type=jnp.float32)


def _scale_body(degp_ref, xw_ref, y1_ref):
    y1_ref[...] = xw_ref[...] * _dinvp(degp_ref)


def _mid_body(degp_ref, acc_ref, y1_ref, w2bd_ref, b1t_ref, h_ref, y2_ref):
    dinvp = _dinvp(degp_ref)
    acc = acc_ref[0, :NP8, :] + acc_ref[1, :NP8, :] + y1_ref[...]
    h = jnp.maximum(acc * dinvp + b1t_ref[0, :][None, :], 0.0)
    h_ref[...] = h
    hw = jnp.dot(h, w2bd_ref[...], preferred_element_type=jnp.float32)
    y2_ref[...] = hw * dinvp


def _fin_body(degp_ref, acc_ref, y2_ref, b2t_ref, z_ref):
    acc = acc_ref[0, :NP8, :] + acc_ref[1, :NP8, :] + y2_ref[...]
    z_ref[...] = acc * _dinvp(degp_ref) + b2t_ref[0, :][None, :]


_mm_call = pl.pallas_call(
    _mm_body, out_shape=jax.ShapeDtypeStruct((NP8, 128), jnp.float32))
_scale_call = pl.pallas_call(
    _scale_body, out_shape=jax.ShapeDtypeStruct((NP8, 128), jnp.float32))
_mid_call = pl.pallas_call(
    _mid_body,
    out_shape=[jax.ShapeDtypeStruct((NP8, 128), jnp.float32),
               jax.ShapeDtypeStruct((NP8, 128), jnp.float32)])
_fin_call = pl.pallas_call(
    _fin_body, out_shape=jax.ShapeDtypeStruct((NP8, 128), jnp.float32))


def kernel(x, edge_index, W1, b1, W2, b2):
    eye8 = jnp.eye(8, dtype=jnp.float32)
    w1bd = jnp.kron(eye8, W1)
    w2p = jnp.zeros((H, H), jnp.float32).at[:, :C].set(W2)
    w2bd = jnp.kron(eye8, w2p)
    b1t = jnp.tile(b1, 8).reshape(1, 128)
    b2t = jnp.tile(jnp.zeros((H,), jnp.float32).at[:C].set(b2), 8)
    b2t = b2t.reshape(1, 128)
    xf = x.reshape(NP8, 8 * D)

    degp = _deg_kernel(edge_index[1])
    xwp = _mm_call(xf, w1bd)
    degp_p = degp.reshape(NC, NPAD8, 128)
    y1p = _scale_call(degp_p, xwp)
    acc1 = _agg_kernel(y1p.reshape(N, H), edge_index)
    h_p, y2p = _mid_call(degp_p, acc1.reshape(NC, NPAD8, 128), y1p,
                         w2bd, b1t)
    acc2 = _agg_kernel(y2p.reshape(N, H), edge_index)
    zp = _fin_call(degp_p, acc2.reshape(NC, NPAD8, 128), y2p, b2t)
    return (h_p.reshape(N, H), zp.reshape(N, H)[:, :C])


# speedup vs baseline: 1.1763x; 1.1763x over previous
import functools

import jax
import jax.numpy as jnp
from jax import lax
from jax.experimental import pallas as pl
from jax.experimental.pallas import tpu as pltpu
from jax.experimental.pallas import tpu_sc as plsc

N = 10000
E = 320000
D = 128
H = 16
C = 7

NC = 2
NS = 16
NW = NC * NS
EW = E // NW
CHUNK = 2000
NCHUNK = EW // CHUNK
NBUF = 3
NPAD = 10240
RPT = NPAD // NS

_mesh = plsc.VectorSubcoreMesh(core_axis_name="c", subcore_axis_name="s")
_sc_params = pltpu.CompilerParams(use_tc_tiling_on_sc=False)


def _fill(ref, n, value):
    v = jnp.full((16,), value, ref.dtype)

    def body(i, _):
        ref[pl.ds(i * 16, 16)] = v
        return 0

    lax.fori_loop(0, n // 16, body, 0)


@functools.partial(
    pl.kernel,
    out_type=jax.ShapeDtypeStruct((NC, NPAD, H), jnp.float32),
    mesh=_mesh,
    scratch_types=[
        pltpu.VMEM((EW,), jnp.int32),
        pltpu.VMEM((EW,), jnp.float32),
        pltpu.VMEM((RPT,), jnp.float32),
        pltpu.VMEM((RPT, H), jnp.float32),
        pltpu.VMEM_SHARED((NPAD,), jnp.float32),
        pltpu.SemaphoreType.DMA,
    ],
    compiler_params=_sc_params,
)
def _deg_kernel(edge_hbm, out_hbm, idx_v, ones_v, buf_v, ebuf_v, deg_sh, sem):
    c = lax.axis_index("c")
    s = lax.axis_index("s")
    wid = s * NC + c
    ld = pltpu.async_copy(edge_hbm.at[1, pl.ds(wid * EW, EW)], idx_v, sem)
    _fill(ones_v, EW, 1.0)
    _fill(buf_v, RPT, 0.0)
    pltpu.sync_copy(buf_v, deg_sh.at[pl.ds(s * RPT, RPT)])
    plsc.subcore_barrier()

    ld.wait()
    pltpu.sync_copy(ones_v, deg_sh.at[idx_v], add=True)

    plsc.subcore_barrier()
    pltpu.sync_copy(deg_sh.at[pl.ds(s * RPT, RPT)], buf_v)

    def expand(g, _):
        vec = buf_v[pl.ds(g * 16, 16)]
        for i in range(16):
            ebuf_v[g * 16 + i, :] = jnp.broadcast_to(vec[i], (16,))
        return 0

    lax.fori_loop(0, RPT // 16, expand, 0)
    pltpu.sync_copy(ebuf_v, out_hbm.at[c, pl.ds(s * RPT, RPT)])


@functools.partial(
    pl.kernel,
    out_type=jax.ShapeDtypeStruct((NC, NPAD, H), jnp.float32),
    mesh=_mesh,
    scratch_types=(
        [pltpu.VMEM((CHUNK,), jnp.int32) for _ in range(NBUF)]
        + [pltpu.VMEM((CHUNK,), jnp.int32) for _ in range(NBUF)]
        + [pltpu.VMEM((CHUNK, H), jnp.float32) for _ in range(NBUF)]
        + [
            pltpu.VMEM((RPT, H), jnp.float32),
            pltpu.VMEM_SHARED((NPAD, H), jnp.float32),
        ]
        + [pltpu.SemaphoreType.DMA for _ in range(3)]
        + [pltpu.SemaphoreType.DMA for _ in range(2)]
        + [pltpu.SemaphoreType.DMA for _ in range(2)]
    ),
    compiler_params=_sc_params,
)
def _agg_kernel(y_hbm, edge_hbm, out_hbm, *refs):
    c = lax.axis_index("c")
    s = lax.axis_index("s")
    wid = s * NC + c
    sidx = refs[0:NBUF]
    didx = refs[NBUF:2 * NBUF]
    rows = refs[2 * NBUF:3 * NBUF]
    buf_v, acc_sh = refs[3 * NBUF], refs[3 * NBUF + 1]
    semI = refs[3 * NBUF + 2:3 * NBUF + 5]
    semG = refs[3 * NBUF + 5:3 * NBUF + 7]
    semS = refs[3 * NBUF + 7:3 * NBUF + 9]

    base = wid * EW

    def load_idx(j):
        sem = semI[j % 3]
        b = j % NBUF
        a1 = pltpu.async_copy(
            edge_hbm.at[0, pl.ds(base + j * CHUNK, CHUNK)], sidx[b], sem)
        a2 = pltpu.async_copy(
            edge_hbm.at[1, pl.ds(base + j * CHUNK, CHUNK)], didx[b], sem)
        return (a1, a2)

    def gather(j):
        return pltpu.async_copy(
            y_hbm.at[sidx[j % NBUF]], rows[j % NBUF], semG[j % 2])

    def scatter(j):
        return pltpu.async_copy(
            rows[j % NBUF], acc_sh.at[didx[j % NBUF]], semS[j % 2], add=True)

    ld = load_idx(0)
    pend_ld = load_idx(1)

    def fill_zero(i, _):
        buf_v[i, :] = jnp.zeros((16,), jnp.float32)
        return 0

    lax.fori_loop(0, RPT, fill_zero, 0)
    pltpu.sync_copy(buf_v, acc_sh.at[pl.ds(s * RPT, RPT)])
    plsc.subcore_barrier()

    ld[0].wait()
    ld[1].wait()
    g = gather(0)
    pend_sc = {}
    for j in range(NCHUNK):
        g.wait()
        pend_sc[j] = scatter(j)
        if j + 2 < NCHUNK:
            if j - 1 >= 0:
                pend_sc.pop(j - 1).wait()
            nxt = load_idx(j + 2)
        if j + 1 < NCHUNK:
            pend_ld[0].wait()
            pend_ld[1].wait()
            g = gather(j + 1)
            pend_ld = nxt
    for j in sorted(pend_sc):
        pend_sc[j].wait()

    plsc.subcore_barrier()
    pltpu.sync_copy(acc_sh.at[pl.ds(s * RPT, RPT)], buf_v)
    pltpu.sync_copy(buf_v, out_hbm.at[c, pl.ds(s * RPT, RPT)])


NP8 = N // 8
NPAD8 = NPAD // 8


def _dinvp(degp_ref):
    deg = degp_ref[0, :NP8, :] + degp_ref[1, :NP8, :] + 1.0
    return lax.rsqrt(deg)


def _mm_body(xf_ref, w1bd_ref, xw_ref):
    xw_ref[...] = jnp.dot(xf_ref[...], w1bd_ref[...],
                          preferred_element_type=jnp.float32)


def _scale_body(degp_ref, xw_ref, y1_ref):
    y1_ref[...] = xw_ref[...] * _dinvp(degp_ref)


def _mid_body(degp_ref, acc_ref, y1_ref, w2bd_ref, b1t_ref, h_ref, y2_ref):
    dinvp = _dinvp(degp_ref)
    acc = acc_ref[0, :NP8, :] + acc_ref[1, :NP8, :] + y1_ref[...]
    h = jnp.maximum(acc * dinvp + b1t_ref[0, :][None, :], 0.0)
    h_ref[...] = h
    hw = jnp.dot(h, w2bd_ref[...], preferred_element_type=jnp.float32)
    y2_ref[...] = hw * dinvp


def _fin_body(degp_ref, acc_ref, y2_ref, b2t_ref, z_ref):
    acc = acc_ref[0, :NP8, :] + acc_ref[1, :NP8, :] + y2_ref[...]
    z_ref[...] = acc * _dinvp(degp_ref) + b2t_ref[0, :][None, :]


_mm_call = pl.pallas_call(
    _mm_body, out_shape=jax.ShapeDtypeStruct((NP8, 128), jnp.float32))
_scale_call = pl.pallas_call(
    _scale_body, out_shape=jax.ShapeDtypeStruct((NP8, 128), jnp.float32))
_mid_call = pl.pallas_call(
    _mid_body,
    out_shape=[jax.ShapeDtypeStruct((NP8, 128), jnp.float32),
               jax.ShapeDtypeStruct((NP8, 128), jnp.float32)])
_fin_call = pl.pallas_call(
    _fin_body, out_shape=jax.ShapeDtypeStruct((NP8, 128), jnp.float32))


def kernel(x, edge_index, W1, b1, W2, b2):
    eye8 = jnp.eye(8, dtype=jnp.float32)
    w1bd = jnp.kron(eye8, W1)
    w2p = jnp.zeros((H, H), jnp.float32).at[:, :C].set(W2)
    w2bd = jnp.kron(eye8, w2p)
    b1t = jnp.tile(b1, 8).reshape(1, 128)
    b2t = jnp.tile(jnp.zeros((H,), jnp.float32).at[:C].set(b2), 8)
    b2t = b2t.reshape(1, 128)
    xf = x.reshape(NP8, 8 * D)

    degp = _deg_kernel(edge_index)
    xwp = _mm_call(xf, w1bd)
    degp_p = degp.reshape(NC, NPAD8, 128)
    y1p = _scale_call(degp_p, xwp)
    acc1 = _agg_kernel(y1p.reshape(N, H), edge_index)
    h_p, y2p = _mid_call(degp_p, acc1.reshape(NC, NPAD8, 128), y1p,
                         w2bd, b1t)
    acc2 = _agg_kernel(y2p.reshape(N, H), edge_index)
    zp = _fin_call(degp_p, acc2.reshape(NC, NPAD8, 128), y2p, b2t)
    return (h_p.reshape(N, H), zp.reshape(N, H)[:, :C])


# speedup vs baseline: 1.2653x; 1.0757x over previous
import functools

import jax
import jax.numpy as jnp
from jax import lax
from jax.experimental import pallas as pl
from jax.experimental.pallas import tpu as pltpu
from jax.experimental.pallas import tpu_sc as plsc

N = 10000
E = 320000
D = 128
H = 16
C = 7

NC = 2
NS = 16
NW = NC * NS
EW = E // NW
CHUNK = 2000
NCHUNK = EW // CHUNK
NBUF = 2
NPAD = 10240
RPT = NPAD // NS

_mesh = plsc.VectorSubcoreMesh(core_axis_name="c", subcore_axis_name="s")
_sc_params = pltpu.CompilerParams(use_tc_tiling_on_sc=False)


def _fill(ref, n, value):
    v = jnp.full((16,), value, ref.dtype)

    def body(i, _):
        ref[pl.ds(i * 16, 16)] = v
        return 0

    lax.fori_loop(0, n // 16, body, 0)


@functools.partial(
    pl.kernel,
    out_type=jax.ShapeDtypeStruct((NC, NPAD, H), jnp.float32),
    mesh=_mesh,
    scratch_types=[
        pltpu.VMEM((EW,), jnp.int32),
        pltpu.VMEM((EW,), jnp.float32),
        pltpu.VMEM((RPT,), jnp.float32),
        pltpu.VMEM((RPT, H), jnp.float32),
        pltpu.VMEM_SHARED((NPAD,), jnp.float32),
        pltpu.SemaphoreType.DMA,
    ],
    compiler_params=_sc_params,
)
def _deg_kernel(edge_hbm, out_hbm, idx_v, ones_v, buf_v, ebuf_v, deg_sh, sem):
    c = lax.axis_index("c")
    s = lax.axis_index("s")
    wid = s * NC + c
    ld = pltpu.async_copy(edge_hbm.at[1, pl.ds(wid * EW, EW)], idx_v, sem)
    _fill(ones_v, EW, 1.0)
    _fill(buf_v, RPT, 0.0)
    pltpu.sync_copy(buf_v, deg_sh.at[pl.ds(s * RPT, RPT)])
    plsc.subcore_barrier()

    ld.wait()
    pltpu.sync_copy(ones_v, deg_sh.at[idx_v], add=True)

    plsc.subcore_barrier()
    pltpu.sync_copy(deg_sh.at[pl.ds(s * RPT, RPT)], buf_v)

    def expand(g, _):
        vec = buf_v[pl.ds(g * 16, 16)]
        for i in range(16):
            ebuf_v[g * 16 + i, :] = jnp.broadcast_to(vec[i], (16,))
        return 0

    lax.fori_loop(0, RPT // 16, expand, 0)
    pltpu.sync_copy(ebuf_v, out_hbm.at[c, pl.ds(s * RPT, RPT)])


@functools.partial(
    pl.kernel,
    out_type=jax.ShapeDtypeStruct((NC, NPAD, H), jnp.float32),
    mesh=_mesh,
    scratch_types=(
        [pltpu.VMEM((CHUNK,), jnp.int32) for _ in range(NBUF)]
        + [pltpu.VMEM((CHUNK,), jnp.int32) for _ in range(NBUF)]
        + [pltpu.VMEM((CHUNK, H), jnp.float32) for _ in range(NBUF)]
        + [
            pltpu.VMEM((RPT, H), jnp.float32),
            pltpu.VMEM_SHARED((NPAD, H), jnp.float32),
            pltpu.VMEM_SHARED((N, H), jnp.float32),
        ]
        + [pltpu.SemaphoreType.DMA for _ in range(3)]
        + [pltpu.SemaphoreType.DMA for _ in range(2)]
        + [pltpu.SemaphoreType.DMA for _ in range(2)]
        + [pltpu.SemaphoreType.DMA]
    ),
    compiler_params=_sc_params,
)
def _agg_kernel(y_hbm, edge_hbm, out_hbm, *refs):
    c = lax.axis_index("c")
    s = lax.axis_index("s")
    wid = s * NC + c
    sidx = refs[0:NBUF]
    didx = refs[NBUF:2 * NBUF]
    rows = refs[2 * NBUF:3 * NBUF]
    buf_v, acc_sh, ytab_sh = refs[3 * NBUF:3 * NBUF + 3]
    semI = refs[3 * NBUF + 3:3 * NBUF + 6]
    semG = refs[3 * NBUF + 6:3 * NBUF + 8]
    semS = refs[3 * NBUF + 8:3 * NBUF + 10]
    semT = refs[3 * NBUF + 10]

    base = wid * EW
    NT = N // NS
    stage = pltpu.async_copy(
        y_hbm.at[pl.ds(s * NT, NT)], ytab_sh.at[pl.ds(s * NT, NT)], semT)

    def load_idx(j):
        sem = semI[j % 3]
        b = j % NBUF
        a1 = pltpu.async_copy(
            edge_hbm.at[0, pl.ds(base + j * CHUNK, CHUNK)], sidx[b], sem)
        a2 = pltpu.async_copy(
            edge_hbm.at[1, pl.ds(base + j * CHUNK, CHUNK)], didx[b], sem)
        return (a1, a2)

    def gather(j):
        return pltpu.async_copy(
            ytab_sh.at[sidx[j % NBUF]], rows[j % NBUF], semG[j % 2])

    def scatter(j):
        return pltpu.async_copy(
            rows[j % NBUF], acc_sh.at[didx[j % NBUF]], semS[j % 2], add=True)

    pend_ld = {0: load_idx(0), 1: load_idx(1)}

    def fill_zero(i, _):
        buf_v[i, :] = jnp.zeros((16,), jnp.float32)
        return 0

    lax.fori_loop(0, RPT, fill_zero, 0)
    pltpu.sync_copy(buf_v, acc_sh.at[pl.ds(s * RPT, RPT)])
    stage.wait()
    plsc.subcore_barrier()

    a1, a2 = pend_ld.pop(0)
    a1.wait()
    a2.wait()
    g = gather(0)
    pend_sc = {}
    for j in range(NCHUNK):
        g.wait()
        pend_sc[j] = scatter(j)
        if j + 1 < NCHUNK:
            a1, a2 = pend_ld.pop(j + 1)
            a1.wait()
            a2.wait()
            g = gather(j + 1)
        if j + 2 < NCHUNK:
            pend_sc.pop(j).wait()
            pend_ld[j + 2] = load_idx(j + 2)
    for j in sorted(pend_sc):
        pend_sc[j].wait()

    plsc.subcore_barrier()
    pltpu.sync_copy(acc_sh.at[pl.ds(s * RPT, RPT)], buf_v)
    pltpu.sync_copy(buf_v, out_hbm.at[c, pl.ds(s * RPT, RPT)])


NP8 = N // 8
NPAD8 = NPAD // 8


def _dinvp(degp_ref):
    deg = degp_ref[0, :NP8, :] + degp_ref[1, :NP8, :] + 1.0
    return lax.rsqrt(deg)


def _mm_body(xf_ref, w1bd_ref, xw_ref):
    xw_ref[...] = jnp.dot(xf_ref[...], w1bd_ref[...],
                          preferred_element_type=jnp.float32)


def _scale_body(degp_ref, xw_ref, y1_ref):
    y1_ref[...] = xw_ref[...] * _dinvp(degp_ref)


def _mid_body(degp_ref, acc_ref, y1_ref, w2bd_ref, b1t_ref, h_ref, y2_ref):
    dinvp = _dinvp(degp_ref)
    acc = acc_ref[0, :NP8, :] + acc_ref[1, :NP8, :] + y1_ref[...]
    h = jnp.maximum(acc * dinvp + b1t_ref[0, :][None, :], 0.0)
    h_ref[...] = h
    hw = jnp.dot(h, w2bd_ref[...], preferred_element_type=jnp.float32)
    y2_ref[...] = hw * dinvp


def _fin_body(degp_ref, acc_ref, y2_ref, b2t_ref, z_ref):
    acc = acc_ref[0, :NP8, :] + acc_ref[1, :NP8, :] + y2_ref[...]
    z_ref[...] = acc * _dinvp(degp_ref) + b2t_ref[0, :][None, :]


_mm_call = pl.pallas_call(
    _mm_body, out_shape=jax.ShapeDtypeStruct((NP8, 128), jnp.float32))
_scale_call = pl.pallas_call(
    _scale_body, out_shape=jax.ShapeDtypeStruct((NP8, 128), jnp.float32))
_mid_call = pl.pallas_call(
    _mid_body,
    out_shape=[jax.ShapeDtypeStruct((NP8, 128), jnp.float32),
               jax.ShapeDtypeStruct((NP8, 128), jnp.float32)])
_fin_call = pl.pallas_call(
    _fin_body, out_shape=jax.ShapeDtypeStruct((NP8, 128), jnp.float32))


def kernel(x, edge_index, W1, b1, W2, b2):
    eye8 = jnp.eye(8, dtype=jnp.float32)
    w1bd = jnp.kron(eye8, W1)
    w2p = jnp.zeros((H, H), jnp.float32).at[:, :C].set(W2)
    w2bd = jnp.kron(eye8, w2p)
    b1t = jnp.tile(b1, 8).reshape(1, 128)
    b2t = jnp.tile(jnp.zeros((H,), jnp.float32).at[:C].set(b2), 8)
    b2t = b2t.reshape(1, 128)
    xf = x.reshape(NP8, 8 * D)

    degp = _deg_kernel(edge_index)
    xwp = _mm_call(xf, w1bd)
    degp_p = degp.reshape(NC, NPAD8, 128)
    y1p = _scale_call(degp_p, xwp)
    acc1 = _agg_kernel(y1p.reshape(N, H), edge_index)
    h_p, y2p = _mid_call(degp_p, acc1.reshape(NC, NPAD8, 128), y1p,
                         w2bd, b1t)
    acc2 = _agg_kernel(y2p.reshape(N, H), edge_index)
    zp = _fin_call(degp_p, acc2.reshape(NC, NPAD8, 128), y2p, b2t)
    return (h_p.reshape(N, H), zp.reshape(N, H)[:, :C])
